# Initial kernel scaffold; baseline (speedup 1.0000x reference)
import jax
import jax.numpy as jnp
from jax.experimental import pallas as pl


def kernel(IOs, table):
    raise NotImplementedError("write your pallas kernel here")


# speedup vs baseline: 1.9806x; 1.9806x over previous
import jax
import jax.numpy as jnp
from jax import lax
from jax.experimental import pallas as pl
from jax.experimental.pallas import tpu as pltpu
from jax.experimental.pallas import tpu_sc as plsc

_B = 1024
_S = 200
_D = 512
_N = _B * _S

_info = plsc.get_sparse_core_info()
_NC, _NS = _info.num_cores, _info.num_subcores
_NW = _NC * _NS
_PER_W = _N // _NW
_CH = 64
_NPAIR = _PER_W // (2 * _CH)


def _body(idx_hbm, table_hbm, out_hbm, idx_v, rows0, rows1, sem0, sem1):
  wid = lax.axis_index("s") * _NC + lax.axis_index("c")
  base = wid * _PER_W
  pltpu.sync_copy(idx_hbm.at[pl.ds(base, _PER_W)], idx_v)

  def gather(c, rows, sem):
    return pltpu.async_copy(
        table_hbm.at[idx_v.at[pl.ds(c * _CH, _CH)]], rows, sem)

  def wait(c, rows, sem):
    pltpu.make_async_copy(
        table_hbm.at[idx_v.at[pl.ds(c * _CH, _CH)]], rows, sem).wait()

  def store(c, rows):
    pltpu.sync_copy(rows, out_hbm.at[pl.ds(base + c * _CH, _CH)])

  gather(0, rows0, sem0)

  def step(i, _):
    c = 2 * i
    gather(c + 1, rows1, sem1)
    wait(c, rows0, sem0)
    store(c, rows0)
    gather(c + 2, rows0, sem0)
    wait(c + 1, rows1, sem1)
    store(c + 1, rows1)
    return _

  lax.fori_loop(0, _NPAIR - 1, step, 0)

  c = 2 * (_NPAIR - 1)
  gather(c + 1, rows1, sem1)
  wait(c, rows0, sem0)
  store(c, rows0)
  wait(c + 1, rows1, sem1)
  store(c + 1, rows1)


_sc_gather = pl.kernel(
    _body,
    out_type=jax.ShapeDtypeStruct((_N, _D), jnp.float32),
    mesh=plsc.VectorSubcoreMesh(core_axis_name="c", subcore_axis_name="s"),
    scratch_types=[
        pltpu.VMEM((_PER_W,), jnp.int32),
        pltpu.VMEM((_CH, _D), jnp.float32),
        pltpu.VMEM((_CH, _D), jnp.float32),
        pltpu.SemaphoreType.DMA,
        pltpu.SemaphoreType.DMA,
    ],
)


@jax.jit
def kernel(IOs, table):
  idx = jnp.transpose(IOs).reshape(-1).astype(jnp.int32)
  out = _sc_gather(idx, table)
  return out.reshape(_S, _B, _D)
